# baseline (speedup 1.0000x reference)
import functools
from typing import Any

import jax
import jax.numpy as jnp
from jax import lax
from jax.experimental import pallas as pl
from jax.experimental.pallas import tpu as pltpu
from jax.experimental.pallas import tpu_sc as plsc

EMBED_DIM = 768
FF_DIM = 3072
NUM_EXPERTS = 8
TOP_K = 2
CAPACITY_FACTOR = 2.0
LOAD_BALANCE_WEIGHT = 0.01
ROUTER_Z_WEIGHT = 0.001

BT = 512
BF = 768


def _sc_gather(table, idx, n_rows, d):
    info = plsc.get_sparse_core_info()
    nw = info.num_cores * info.num_subcores
    bpw = n_rows // nw
    nchunk = 2
    while (bpw // nchunk) * d * 4 * 2 > 440_000 or bpw % nchunk:
        nchunk += 2
    ch = bpw // nchunk
    mesh = plsc.VectorSubcoreMesh(core_axis_name="c", subcore_axis_name="s")

    @functools.partial(
        pl.kernel, mesh=mesh,
        out_type=jax.ShapeDtypeStruct((n_rows, d), jnp.float32),
        scratch_types=[
            pltpu.VMEM((bpw,), jnp.int32),
            pltpu.VMEM((ch, d), jnp.float32),
            pltpu.VMEM((ch, d), jnp.float32),
            pltpu.SemaphoreType.DMA,
            pltpu.SemaphoreType.DMA,
            pltpu.SemaphoreType.DMA,
            pltpu.SemaphoreType.DMA,
        ],
    )
    def gather_kernel(table_hbm, idx_hbm, out_hbm, idx_v, buf_a, buf_b,
                      gs_a, gs_b, ss_a, ss_b):
        wid = lax.axis_index("s") * info.num_cores + lax.axis_index("c")
        base = wid * bpw
        pltpu.sync_copy(idx_hbm.at[pl.ds(base, bpw)], idx_v)
        bufs = (buf_a, buf_b)
        gsems = (gs_a, gs_b)
        ssems = (ss_a, ss_b)
        g = [None] * nchunk
        s = [None] * nchunk
        for c in range(nchunk):
            p = c % 2
            if c >= 2:
                s[c - 2].wait()
            g[c] = pltpu.async_copy(
                table_hbm.at[idx_v.at[pl.ds(c * ch, ch)]], bufs[p], gsems[p])
            if c >= 1:
                g[c - 1].wait()
                s[c - 1] = pltpu.async_copy(
                    bufs[(c - 1) % 2],
                    out_hbm.at[pl.ds(base + (c - 1) * ch, ch)],
                    ssems[(c - 1) % 2])
        g[nchunk - 1].wait()
        s[nchunk - 1] = pltpu.async_copy(
            bufs[(nchunk - 1) % 2],
            out_hbm.at[pl.ds(base + (nchunk - 1) * ch, ch)],
            ssems[(nchunk - 1) % 2])
        if nchunk >= 2:
            s[nchunk - 2].wait()
        s[nchunk - 1].wait()

    return gather_kernel(table, idx)


def _ffn_body(bexp_ref, nb_ref, x_ref, wg_ref, wu_ref, wo_ref, y_ref):
    b = pl.program_id(0)
    j = pl.program_id(1)
    active = b < nb_ref[0]

    @pl.when(active)
    def _compute():
        xb = x_ref[:, :]
        wg = wg_ref[0]
        wu = wu_ref[0]
        wo = wo_ref[0]
        dn = (((1,), (1,)), ((), ()))
        g = jax.lax.dot_general(xb, wg, dn, preferred_element_type=jnp.float32)
        u = jax.lax.dot_general(xb, wu, dn, preferred_element_type=jnp.float32)
        h = (g * jax.nn.sigmoid(g)) * u
        yb = jax.lax.dot_general(h, wo, dn, preferred_element_type=jnp.float32)

        @pl.when(j == 0)
        def _():
            y_ref[:, :] = yb

        @pl.when(j > 0)
        def _():
            y_ref[:, :] = y_ref[:, :] + yb

    @pl.when(jnp.logical_not(active) & (j == 0))
    def _zero():
        y_ref[:, :] = jnp.zeros_like(y_ref)


def _grouped_ffn(xg, bexp, nb, wi_gate, wi_up, wo, gmax, nbmax):
    D = xg.shape[1]
    nf = FF_DIM // BF
    grid_spec = pltpu.PrefetchScalarGridSpec(
        num_scalar_prefetch=2,
        grid=(nbmax, nf),
        in_specs=[
            pl.BlockSpec((BT, D), lambda b, j, bexp, nb: (b, 0)),
            pl.BlockSpec((1, BF, D), lambda b, j, bexp, nb: (bexp[b], j, 0)),
            pl.BlockSpec((1, BF, D), lambda b, j, bexp, nb: (bexp[b], j, 0)),
            pl.BlockSpec((1, D, BF), lambda b, j, bexp, nb: (bexp[b], 0, j)),
        ],
        out_specs=pl.BlockSpec((BT, D), lambda b, j, bexp, nb: (b, 0)),
    )
    return pl.pallas_call(
        _ffn_body,
        grid_spec=grid_spec,
        out_shape=jax.ShapeDtypeStruct((gmax, D), jnp.float32),
    )(bexp, nb, xg, wi_gate, wi_up, wo)


def kernel(x, gate_w, wi_gate, wi_up, wo):
    B, S, D = x.shape
    T = B * S
    E = NUM_EXPERTS
    cap = max(int(T * TOP_K / E * CAPACITY_FACTOR), TOP_K)
    xf = x.reshape(T, D)

    logits = xf @ gate_w.T
    probs = jax.nn.softmax(logits, axis=-1)
    i0 = jnp.argmax(probs, axis=-1)
    p0 = jnp.max(probs, axis=-1)
    e_ids = jnp.arange(E, dtype=jnp.int32)
    masked = jnp.where(i0[:, None] == e_ids[None, :], -jnp.inf, probs)
    i1 = jnp.argmax(masked, axis=-1)
    p1 = jnp.max(masked, axis=-1)
    s = p0 + p1
    w0 = p0 / s
    w1 = p1 / s

    oh0 = (i0[:, None] == e_ids[None, :]).astype(jnp.int32)
    oh1 = (i1[:, None] == e_ids[None, :]).astype(jnp.int32)
    cum0 = jnp.cumsum(oh0, axis=0)
    cum1 = jnp.cumsum(oh1, axis=0)
    rank0 = jnp.sum(cum0 * oh0, axis=1)
    rank1 = jnp.sum(cum1 * oh1, axis=1)
    kept0 = rank0 <= cap
    kept1 = rank1 <= cap
    cnt0 = jnp.sum((cum0 <= cap) * oh0, axis=0)
    cnt1 = jnp.sum((cum1 <= cap) * oh1, axis=0)
    size = cnt0 + cnt1
    padded = ((size + BT - 1) // BT) * BT
    off = jnp.concatenate([jnp.zeros((1,), jnp.int32),
                           jnp.cumsum(padded)[:-1].astype(jnp.int32)])
    nb = jnp.sum(padded, dtype=jnp.int32) // BT

    gmax = T * TOP_K + E * BT
    nbmax = gmax // BT

    tarange = jnp.arange(T, dtype=jnp.int32)
    row0 = jnp.where(kept0, off[i0] + rank0 - 1, gmax)
    row1 = jnp.where(kept1, off[i1] + cnt0[i1] + rank1 - 1, gmax)
    tok_ids = jnp.zeros((gmax + 1,), jnp.int32)
    tok_ids = tok_ids.at[row0].set(tarange).at[row1].set(tarange)
    tok_ids = tok_ids[:gmax]

    blk_start = off // BT
    barange = jnp.arange(nbmax, dtype=jnp.int32)
    bexp = jnp.searchsorted(blk_start, barange, side='right').astype(jnp.int32) - 1
    last = jnp.maximum(nb - 1, 0)
    bexp = jnp.where(barange < nb, bexp, bexp[last])
    bexp = jnp.clip(bexp, 0, E - 1)

    xg = _sc_gather(xf, tok_ids, gmax, D)

    y = _grouped_ffn(xg, bexp, nb.reshape(1),
                     wi_gate, wi_up, wo, gmax, nbmax)

    r0 = jnp.where(kept0, row0, 0)
    r1 = jnp.where(kept1, row1, 0)
    w0k = jnp.where(kept0, w0, 0.0)
    w1k = jnp.where(kept1, w1, 0.0)
    out = y[r0] * w0k[:, None] + y[r1] * w1k[:, None]
    output = out.reshape(B, S, D)

    f = (oh0 + oh1).sum(axis=0).astype(jnp.float32) / (T * TOP_K)
    P = probs.mean(axis=0)
    load_balance_loss = E * jnp.sum(f * P)
    lse = jax.scipy.special.logsumexp(logits, axis=-1)
    z_loss = jnp.mean(jnp.square(lse))
    aux_loss = (LOAD_BALANCE_WEIGHT * load_balance_loss
                + ROUTER_Z_WEIGHT * z_loss)
    return (output, aux_loss)


# speedup vs baseline: 1.8862x; 1.8862x over previous
import jax
import jax.numpy as jnp
from jax import lax
from jax.experimental import pallas as pl
from jax.experimental.pallas import tpu as pltpu

EMBED_DIM = 768
FF_DIM = 3072
NUM_EXPERTS = 8
TOP_K = 2
CAPACITY_FACTOR = 2.0
LOAD_BALANCE_WEIGHT = 0.01
ROUTER_Z_WEIGHT = 0.001

BT = 512
BF = 768
NF = FF_DIM // BF


def _ffn_body(offs_ref, nbe_ref, x_ref, r0_ref, r1_ref, w0_ref, w1_ref,
              wg_ref, wu_ref, wo_ref, out_ref, xg_ref, yacc_ref):
    e = pl.program_id(0)
    b = pl.program_id(1)
    j = pl.program_id(2)
    active = b < nbe_ref[e]
    base = offs_ref[e] + b * BT
    T = x_ref.shape[0]

    @pl.when((e == 0) & (b == 0) & (j == 0))
    def _init():
        out_ref[:, :] = jnp.zeros_like(out_ref)

    @pl.when(active)
    def _compute():
        @pl.when(j == 0)
        def _gather():
            rows = lax.broadcasted_iota(jnp.int32, (BT, T), 0) + base
            m = ((rows == r0_ref[0][None, :]).astype(jnp.float32)
                 + (rows == r1_ref[0][None, :]).astype(jnp.float32))
            xg_ref[:, :] = jnp.dot(m, x_ref[:, :],
                                   preferred_element_type=jnp.float32)

        xb = xg_ref[:, :]
        wg = wg_ref[0]
        wu = wu_ref[0]
        wo = wo_ref[0]
        dn = (((1,), (1,)), ((), ()))
        g = lax.dot_general(xb, wg, dn, preferred_element_type=jnp.float32)
        u = lax.dot_general(xb, wu, dn, preferred_element_type=jnp.float32)
        h = (g * jax.nn.sigmoid(g)) * u
        yb = lax.dot_general(h, wo, dn, preferred_element_type=jnp.float32)

        @pl.when(j == 0)
        def _():
            yacc_ref[:, :] = yb

        @pl.when(j > 0)
        def _():
            yacc_ref[:, :] = yacc_ref[:, :] + yb

        @pl.when(j == NF - 1)
        def _scatter():
            rows = lax.broadcasted_iota(jnp.int32, (BT, T), 0) + base
            mw = ((rows == r0_ref[0][None, :]).astype(jnp.float32)
                  * w0_ref[0][None, :]
                  + (rows == r1_ref[0][None, :]).astype(jnp.float32)
                  * w1_ref[0][None, :])
            dns = (((0,), (0,)), ((), ()))
            out_ref[:, :] = out_ref[:, :] + lax.dot_general(
                mw, yacc_ref[:, :], dns, preferred_element_type=jnp.float32)


def _moe_ffn(xf, row0, row1, w0k, w1k, offs, nbe, wi_gate, wi_up, wo, nbe_max):
    T, D = xf.shape

    def jeff(b, j, nbe, e):
        return jnp.where(b < nbe[e], j, NF - 1)

    grid_spec = pltpu.PrefetchScalarGridSpec(
        num_scalar_prefetch=2,
        grid=(NUM_EXPERTS, nbe_max, NF),
        in_specs=[
            pl.BlockSpec((T, D), lambda e, b, j, offs, nbe: (0, 0)),
            pl.BlockSpec((1, T), lambda e, b, j, offs, nbe: (0, 0)),
            pl.BlockSpec((1, T), lambda e, b, j, offs, nbe: (0, 0)),
            pl.BlockSpec((1, T), lambda e, b, j, offs, nbe: (0, 0)),
            pl.BlockSpec((1, T), lambda e, b, j, offs, nbe: (0, 0)),
            pl.BlockSpec((1, BF, D),
                         lambda e, b, j, offs, nbe: (e, jeff(b, j, nbe, e), 0)),
            pl.BlockSpec((1, BF, D),
                         lambda e, b, j, offs, nbe: (e, jeff(b, j, nbe, e), 0)),
            pl.BlockSpec((1, D, BF),
                         lambda e, b, j, offs, nbe: (e, 0, jeff(b, j, nbe, e))),
        ],
        out_specs=pl.BlockSpec((T, D), lambda e, b, j, offs, nbe: (0, 0)),
        scratch_shapes=[
            pltpu.VMEM((BT, D), jnp.float32),
            pltpu.VMEM((BT, D), jnp.float32),
        ],
    )
    return pl.pallas_call(
        _ffn_body,
        grid_spec=grid_spec,
        out_shape=jax.ShapeDtypeStruct((T, D), jnp.float32),
    )(offs, nbe, xf, row0.reshape(1, T), row1.reshape(1, T),
      w0k.reshape(1, T), w1k.reshape(1, T), wi_gate, wi_up, wo)


def kernel(x, gate_w, wi_gate, wi_up, wo):
    B, S, D = x.shape
    T = B * S
    E = NUM_EXPERTS
    cap = max(int(T * TOP_K / E * CAPACITY_FACTOR), TOP_K)
    xf = x.reshape(T, D)

    logits = xf @ gate_w.T
    probs = jax.nn.softmax(logits, axis=-1)
    i0 = jnp.argmax(probs, axis=-1)
    p0 = jnp.max(probs, axis=-1)
    e_ids = jnp.arange(E, dtype=jnp.int32)
    masked = jnp.where(i0[:, None] == e_ids[None, :], -jnp.inf, probs)
    i1 = jnp.argmax(masked, axis=-1)
    p1 = jnp.max(masked, axis=-1)
    s = p0 + p1
    w0 = p0 / s
    w1 = p1 / s

    oh0 = (i0[:, None] == e_ids[None, :]).astype(jnp.int32)
    oh1 = (i1[:, None] == e_ids[None, :]).astype(jnp.int32)
    cum0 = jnp.cumsum(oh0, axis=0)
    cum1 = jnp.cumsum(oh1, axis=0)
    rank0 = jnp.sum(cum0 * oh0, axis=1)
    rank1 = jnp.sum(cum1 * oh1, axis=1)
    kept0 = rank0 <= cap
    kept1 = rank1 <= cap
    cnt0 = jnp.sum((cum0 <= cap) * oh0, axis=0)
    cnt1 = jnp.sum((cum1 <= cap) * oh1, axis=0)
    size = cnt0 + cnt1
    padded = ((size + BT - 1) // BT) * BT
    offs = jnp.concatenate([jnp.zeros((1,), jnp.int32),
                            jnp.cumsum(padded)[:-1].astype(jnp.int32)])
    nbe = (padded // BT).astype(jnp.int32)
    nbe_max = (2 * cap + BT - 1) // BT

    sentinel = T * TOP_K + E * BT
    row0 = jnp.where(kept0, offs[i0] + rank0 - 1, sentinel).astype(jnp.int32)
    row1 = jnp.where(kept1, offs[i1] + cnt0[i1] + rank1 - 1,
                     sentinel).astype(jnp.int32)
    w0k = jnp.where(kept0, w0, 0.0)
    w1k = jnp.where(kept1, w1, 0.0)

    out = _moe_ffn(xf, row0, row1, w0k, w1k, offs, nbe,
                   wi_gate, wi_up, wo, nbe_max)
    output = out.reshape(B, S, D)

    f = (oh0 + oh1).sum(axis=0).astype(jnp.float32) / (T * TOP_K)
    P = probs.mean(axis=0)
    load_balance_loss = E * jnp.sum(f * P)
    lse = jax.scipy.special.logsumexp(logits, axis=-1)
    z_loss = jnp.mean(jnp.square(lse))
    aux_loss = (LOAD_BALANCE_WEIGHT * load_balance_loss
                + ROUTER_Z_WEIGHT * z_loss)
    return (output, aux_loss)
